# baseline (speedup 1.0000x reference)
import functools
import jax
import jax.numpy as jnp
import numpy as np
from jax.experimental import pallas as pl
from jax.experimental.pallas import tpu as pltpu

B = 8
P = 2048
K = 30

_HI = jax.lax.Precision.HIGHEST


def _dot(a, b, precision=None):
    return jnp.dot(a, b, preferred_element_type=jnp.float32, precision=precision)


def _dot_hi(a, b):
    return jnp.dot(a, b, preferred_element_type=jnp.float32, precision=_HI)


def _gn_mats(C):
    groups = max(1, C // 16)
    gs = C // groups
    gid = np.arange(C) // gs
    M = np.zeros((C, groups), np.float32)
    M[np.arange(C), gid] = 1.0 / gs
    S = (M.T > 0).astype(np.float32)
    return jnp.asarray(M), jnp.asarray(S)


def _tower_params(layers):
    flat = []
    for (W, b, g, be) in layers:
        C = W.shape[0]
        M, S = _gn_mats(C)
        flat += [jnp.asarray(W.T), b[None, :], g[None, :], be[None, :], M, S]
    return flat


def _tfold(xg):
    s = xg[..., :8] + xg[..., 8:]
    s = s[..., :4] + s[..., 4:]
    s = s[..., :2] + s[..., 2:]
    return (s[..., :1] + s[..., 1:]) * (1.0 / 16.0)


def _apply_tower(x, wrefs, nl, treefold=False):
    for l in range(nl):
        Wt, b, g, be, M, S = (r[...] for r in wrefs[6 * l:6 * l + 6])
        x = _dot(x, Wt) + b
        x = jnp.maximum(x, 0.0)
        R, C = x.shape
        if treefold and C % 16 == 0:
            xg = x.reshape(R, C // 16, 16)
            mu = _tfold(xg)
            d = xg - mu
            var = _tfold(d * d)
            x = (d * jax.lax.rsqrt(var + 1e-5)).reshape(R, C) * g + be
        else:
            mu = _dot_hi(_dot_hi(x, M), S)
            d = x - mu
            var = _dot_hi(_dot_hi(d * d, M), S)
            x = d * jax.lax.rsqrt(var + 1e-5) * g + be
    return x


def _full_spec(w):
    return pl.BlockSpec(w.shape, lambda *i, nd=w.ndim: (0,) * nd)


def _edge_conv(x, xj_flat, layers, T=128, treefold=False):
    Npts, Cin = x.shape
    nl = len(layers)
    Cout = layers[-1][0].shape[0]
    wflat = _tower_params(layers)

    def body(*refs):
        xi_ref, xj_ref = refs[0], refs[1]
        wrefs = refs[2:2 + 6 * nl]
        o_ref = refs[-1]
        xi = xi_ref[...]
        xj = xj_ref[...]
        xi_rep = jnp.repeat(xi, K, axis=0)
        e = jnp.concatenate([xi_rep, xj - xi_rep], axis=-1)
        e = _apply_tower(e, wrefs, nl, treefold=treefold)
        o_ref[...] = jnp.max(e.reshape(T, K, Cout), axis=1)

    in_specs = [pl.BlockSpec((T, Cin), lambda i: (i, 0)),
                pl.BlockSpec((T * K, Cin), lambda i: (i, 0))]
    in_specs += [_full_spec(w) for w in wflat]
    return pl.pallas_call(
        body,
        grid=(Npts // T,),
        in_specs=in_specs,
        out_specs=pl.BlockSpec((T, Cout), lambda i: (i, 0)),
        out_shape=jax.ShapeDtypeStruct((Npts, Cout), jnp.float32),
        compiler_params=pltpu.CompilerParams(
            vmem_limit_bytes=100 * 1024 * 1024),
    )(x, xj_flat, *wflat)


def _mlp_pool(x, layers, nb, T=256, treefold=False):
    Npts, Cin = x.shape
    pp = Npts // nb
    nl = len(layers)
    Cout = layers[-1][0].shape[0]
    wflat = _tower_params(layers)

    def body(*refs):
        x_ref = refs[0]
        wrefs = refs[1:1 + 6 * nl]
        o_ref = refs[-1]
        y = _apply_tower(x_ref[...], wrefs, nl, treefold=treefold)
        m = jnp.max(y, axis=0, keepdims=True)[None]
        j = pl.program_id(1)

        @pl.when(j == 0)
        def _():
            o_ref[...] = m

        @pl.when(j != 0)
        def _():
            o_ref[...] = jnp.maximum(o_ref[...], m)

    in_specs = [pl.BlockSpec((T, Cin), lambda b, j: (b * (pp // T) + j, 0))]
    in_specs += [_full_spec(w) for w in wflat]
    out = pl.pallas_call(
        body,
        grid=(nb, pp // T),
        in_specs=in_specs,
        out_specs=pl.BlockSpec((1, 1, Cout), lambda b, j: (b, 0, 0)),
        out_shape=jax.ShapeDtypeStruct((nb, 1, Cout), jnp.float32),
        compiler_params=pltpu.CompilerParams(
            vmem_limit_bytes=100 * 1024 * 1024),
    )(x, *wflat)
    return out.reshape(nb, Cout)


def _mlp_plain(x, layers, treefold=False):
    R, Cin = x.shape
    nl = len(layers)
    Cout = layers[-1][0].shape[0]
    wflat = _tower_params(layers)

    def body(*refs):
        x_ref = refs[0]
        wrefs = refs[1:1 + 6 * nl]
        o_ref = refs[-1]
        o_ref[...] = _apply_tower(x_ref[...], wrefs, nl, treefold=treefold)

    in_specs = [_full_spec(x)] + [_full_spec(w) for w in wflat]
    return pl.pallas_call(
        body,
        grid=(1,),
        in_specs=in_specs,
        out_specs=pl.BlockSpec((R, Cout), lambda i: (0, 0)),
        out_shape=jax.ShapeDtypeStruct((R, Cout), jnp.float32),
    )(x, *wflat)


def _transform(pos, t4, nb):
    Npts = pos.shape[0]
    pp = Npts // nb
    pos3 = pos.reshape(nb, pp, 3)
    mats = t4.reshape(nb, 3, 3)

    def body(x_ref, m_ref, o_ref):
        o_ref[0] = _dot(x_ref[0], m_ref[0])

    out = pl.pallas_call(
        body,
        grid=(nb,),
        in_specs=[pl.BlockSpec((1, pp, 3), lambda b: (b, 0, 0)),
                  pl.BlockSpec((1, 3, 3), lambda b: (b, 0, 0))],
        out_specs=pl.BlockSpec((1, pp, 3), lambda b: (b, 0, 0)),
        out_shape=jax.ShapeDtypeStruct((nb, pp, 3), jnp.float32),
    )(pos3, mats)
    return out.reshape(Npts, 3)


def _head(x2, x3, x5, head_layers, lin2_W, lin2_b, lin3_W, lin3_b, nb, T=256):
    Npts, C2 = x2.shape
    pp = Npts // nb
    C3 = x3.shape[1]
    C5 = x5.shape[1]
    nl = len(head_layers)
    wflat = _tower_params(head_layers)
    wflat += [jnp.asarray(lin2_W.T), lin2_b[None, :],
              jnp.asarray(lin3_W.T), lin3_b[None, :]]
    Cout = lin3_W.shape[0]

    def body(*refs):
        x2_ref, x3_ref, x5_ref = refs[0], refs[1], refs[2]
        wrefs = refs[3:3 + 6 * nl]
        l2w, l2b, l3w, l3b = (r[...] for r in refs[3 + 6 * nl:3 + 6 * nl + 4])
        o_ref = refs[-1]
        x6 = jnp.broadcast_to(x5_ref[0], (T, C5))
        x = jnp.concatenate([x2_ref[...], x3_ref[...], x6], axis=-1)
        h = _apply_tower(x, wrefs, nl)
        h = _dot(h, l2w) + l2b
        h = _dot(h, l3w) + l3b
        m = jnp.max(h, axis=-1, keepdims=True)
        s = h - m
        lse = jnp.log(jnp.sum(jnp.exp(s), axis=-1, keepdims=True))
        o_ref[...] = s - lse

    row_spec = lambda C: pl.BlockSpec((T, C), lambda b, j: (b * (pp // T) + j, 0))
    in_specs = [row_spec(C2), row_spec(C3),
                pl.BlockSpec((1, 1, C5), lambda b, j: (b, 0, 0))]
    in_specs += [_full_spec(w) for w in wflat]
    return pl.pallas_call(
        body,
        grid=(nb, pp // T),
        in_specs=in_specs,
        out_specs=pl.BlockSpec((T, Cout), lambda b, j: (b * (pp // T) + j, 0)),
        out_shape=jax.ShapeDtypeStruct((Npts, Cout), jnp.float32),
    )(x2, x3, x5.reshape(nb, 1, C5), *wflat)


def _knn_gather(x, nb, k=K):
    Npts, C = x.shape
    pp = Npts // nb
    xb = x.reshape(nb, pp, C)
    sq = jnp.sum(xb * xb, axis=-1)
    d = sq[:, :, None] + sq[:, None, :] - 2.0 * jnp.einsum('bic,bjc->bij', xb, xb)
    _, idx = jax.lax.top_k(-d, k)
    bidx = jnp.arange(nb)[:, None, None]
    xj = xb[bidx, idx]
    return xj.reshape(Npts * k, C)


def _final_gn_layer(x, layer):
    W, b, g, be = layer
    x = jax.nn.relu(x @ W.T + b)
    Nr, C = x.shape
    mean = x.mean(axis=-1, keepdims=True)
    var = ((x - mean) ** 2).mean(axis=-1, keepdims=True)
    return (x - mean) * jax.lax.rsqrt(var + 1e-5) * g + be


def _forward(pos, params, nb):
    xj0 = _knn_gather(pos, nb)
    t1 = _edge_conv(pos, xj0, params['tconv1'], treefold=True)
    t3 = _mlp_pool(t1, params['tmlp1'], nb, treefold=True)
    t4 = _mlp_plain(t3, params['tmlp2'][:-1], treefold=True)
    t4 = _final_gn_layer(t4, params['tmlp2'][-1])
    pos2 = _transform(pos, t4, nb)
    x1 = _edge_conv(pos2, _knn_gather(pos2, nb), params['conv1'], treefold=True)
    x2 = _edge_conv(x1, _knn_gather(x1, nb), params['conv2'], treefold=True)
    x3 = _edge_conv(x2, _knn_gather(x2, nb), params['conv3'])
    x5 = _mlp_pool(x3, params['lin1'], nb)
    return _head(x2, x3, x5, params['head'],
                 params['lin2_W'], params['lin2_b'],
                 params['lin3_W'], params['lin3_b'], nb)


def kernel(pos, batch, params):
    del batch
    return _forward(pos, params, B)


# speedup vs baseline: 1.0449x; 1.0449x over previous
import functools
import jax
import jax.numpy as jnp
import numpy as np
from jax.experimental import pallas as pl
from jax.experimental.pallas import tpu as pltpu

B = 8
P = 2048
K = 30

_HI = jax.lax.Precision.HIGHEST


def _dot(a, b, precision=None):
    return jnp.dot(a, b, preferred_element_type=jnp.float32, precision=precision)


def _dot_hi(a, b):
    return jnp.dot(a, b, preferred_element_type=jnp.float32, precision=_HI)


def _gn_mats(C):
    groups = max(1, C // 16)
    gs = C // groups
    gid = np.arange(C) // gs
    M = np.zeros((C, groups), np.float32)
    M[np.arange(C), gid] = 1.0 / gs
    S = (M.T > 0).astype(np.float32)
    return jnp.asarray(M), jnp.asarray(S)


def _tower_params(layers):
    flat = []
    for (W, b, g, be) in layers:
        C = W.shape[0]
        M, S = _gn_mats(C)
        flat += [jnp.asarray(W.T), b[None, :], g[None, :], be[None, :], M, S]
    return flat


def _tfold(xg):
    s = xg[..., :8] + xg[..., 8:]
    s = s[..., :4] + s[..., 4:]
    s = s[..., :2] + s[..., 2:]
    return (s[..., :1] + s[..., 1:]) * (1.0 / 16.0)


def _apply_tower(x, wrefs, nl, treefold=False):
    for l in range(nl):
        Wt, b, g, be, M, S = (r[...] for r in wrefs[6 * l:6 * l + 6])
        x = _dot(x, Wt) + b
        x = jnp.maximum(x, 0.0)
        R, C = x.shape
        if treefold and C % 16 == 0:
            xg = x.reshape(R, C // 16, 16)
            mu = _tfold(xg)
            d = xg - mu
            var = _tfold(d * d)
            x = (d * jax.lax.rsqrt(var + 1e-5)).reshape(R, C) * g + be
        else:
            mu = _dot(_dot(x, M), S)
            d = x - mu
            var = _dot(_dot(d * d, M), S)
            x = d * jax.lax.rsqrt(var + 1e-5) * g + be
    return x


def _full_spec(w):
    return pl.BlockSpec(w.shape, lambda *i, nd=w.ndim: (0,) * nd)


def _edge_conv(x, xj_flat, layers, T=128, treefold=False):
    Npts, Cin = x.shape
    nl = len(layers)
    Cout = layers[-1][0].shape[0]
    wflat = _tower_params(layers)

    def body(*refs):
        xi_ref, xj_ref = refs[0], refs[1]
        wrefs = refs[2:2 + 6 * nl]
        o_ref = refs[-1]
        xi = xi_ref[...]
        xj = xj_ref[...].reshape(K * T, Cin)
        xi_rep = jnp.broadcast_to(xi[None], (K, T, Cin)).reshape(K * T, Cin)
        e = jnp.concatenate([xi_rep, xj - xi_rep], axis=-1)
        e = _apply_tower(e, wrefs, nl, treefold=treefold)
        acc = e[0:T]
        for k in range(1, K):
            acc = jnp.maximum(acc, e[k * T:(k + 1) * T])
        o_ref[...] = acc

    in_specs = [pl.BlockSpec((T, Cin), lambda i: (i, 0)),
                pl.BlockSpec((K, T, Cin), lambda i: (0, i, 0))]
    in_specs += [_full_spec(w) for w in wflat]
    return pl.pallas_call(
        body,
        grid=(Npts // T,),
        in_specs=in_specs,
        out_specs=pl.BlockSpec((T, Cout), lambda i: (i, 0)),
        out_shape=jax.ShapeDtypeStruct((Npts, Cout), jnp.float32),
        compiler_params=pltpu.CompilerParams(
            vmem_limit_bytes=100 * 1024 * 1024),
    )(x, xj_flat, *wflat)


def _mlp_pool(x, layers, nb, T=256, treefold=False):
    Npts, Cin = x.shape
    pp = Npts // nb
    nl = len(layers)
    Cout = layers[-1][0].shape[0]
    wflat = _tower_params(layers)

    def body(*refs):
        x_ref = refs[0]
        wrefs = refs[1:1 + 6 * nl]
        o_ref = refs[-1]
        y = _apply_tower(x_ref[...], wrefs, nl, treefold=treefold)
        m = jnp.max(y, axis=0, keepdims=True)[None]
        j = pl.program_id(1)

        @pl.when(j == 0)
        def _():
            o_ref[...] = m

        @pl.when(j != 0)
        def _():
            o_ref[...] = jnp.maximum(o_ref[...], m)

    in_specs = [pl.BlockSpec((T, Cin), lambda b, j: (b * (pp // T) + j, 0))]
    in_specs += [_full_spec(w) for w in wflat]
    out = pl.pallas_call(
        body,
        grid=(nb, pp // T),
        in_specs=in_specs,
        out_specs=pl.BlockSpec((1, 1, Cout), lambda b, j: (b, 0, 0)),
        out_shape=jax.ShapeDtypeStruct((nb, 1, Cout), jnp.float32),
        compiler_params=pltpu.CompilerParams(
            vmem_limit_bytes=100 * 1024 * 1024),
    )(x, *wflat)
    return out.reshape(nb, Cout)


def _mlp_plain(x, layers, treefold=False):
    R, Cin = x.shape
    nl = len(layers)
    Cout = layers[-1][0].shape[0]
    wflat = _tower_params(layers)

    def body(*refs):
        x_ref = refs[0]
        wrefs = refs[1:1 + 6 * nl]
        o_ref = refs[-1]
        o_ref[...] = _apply_tower(x_ref[...], wrefs, nl, treefold=treefold)

    in_specs = [_full_spec(x)] + [_full_spec(w) for w in wflat]
    return pl.pallas_call(
        body,
        grid=(1,),
        in_specs=in_specs,
        out_specs=pl.BlockSpec((R, Cout), lambda i: (0, 0)),
        out_shape=jax.ShapeDtypeStruct((R, Cout), jnp.float32),
    )(x, *wflat)


def _transform(pos, t4, nb):
    Npts = pos.shape[0]
    pp = Npts // nb
    pos3 = pos.reshape(nb, pp, 3)
    mats = t4.reshape(nb, 3, 3)

    def body(x_ref, m_ref, o_ref):
        o_ref[0] = _dot(x_ref[0], m_ref[0])

    out = pl.pallas_call(
        body,
        grid=(nb,),
        in_specs=[pl.BlockSpec((1, pp, 3), lambda b: (b, 0, 0)),
                  pl.BlockSpec((1, 3, 3), lambda b: (b, 0, 0))],
        out_specs=pl.BlockSpec((1, pp, 3), lambda b: (b, 0, 0)),
        out_shape=jax.ShapeDtypeStruct((nb, pp, 3), jnp.float32),
    )(pos3, mats)
    return out.reshape(Npts, 3)


def _head(x2, x3, x5, head_layers, lin2_W, lin2_b, lin3_W, lin3_b, nb, T=256):
    Npts, C2 = x2.shape
    pp = Npts // nb
    C3 = x3.shape[1]
    C5 = x5.shape[1]
    nl = len(head_layers)
    wflat = _tower_params(head_layers)
    wflat += [jnp.asarray(lin2_W.T), lin2_b[None, :],
              jnp.asarray(lin3_W.T), lin3_b[None, :]]
    Cout = lin3_W.shape[0]

    def body(*refs):
        x2_ref, x3_ref, x5_ref = refs[0], refs[1], refs[2]
        wrefs = refs[3:3 + 6 * nl]
        l2w, l2b, l3w, l3b = (r[...] for r in refs[3 + 6 * nl:3 + 6 * nl + 4])
        o_ref = refs[-1]
        x6 = jnp.broadcast_to(x5_ref[0], (T, C5))
        x = jnp.concatenate([x2_ref[...], x3_ref[...], x6], axis=-1)
        h = _apply_tower(x, wrefs, nl)
        h = _dot(h, l2w) + l2b
        h = _dot(h, l3w) + l3b
        m = jnp.max(h, axis=-1, keepdims=True)
        s = h - m
        lse = jnp.log(jnp.sum(jnp.exp(s), axis=-1, keepdims=True))
        o_ref[...] = s - lse

    row_spec = lambda C: pl.BlockSpec((T, C), lambda b, j: (b * (pp // T) + j, 0))
    in_specs = [row_spec(C2), row_spec(C3),
                pl.BlockSpec((1, 1, C5), lambda b, j: (b, 0, 0))]
    in_specs += [_full_spec(w) for w in wflat]
    return pl.pallas_call(
        body,
        grid=(nb, pp // T),
        in_specs=in_specs,
        out_specs=pl.BlockSpec((T, Cout), lambda b, j: (b * (pp // T) + j, 0)),
        out_shape=jax.ShapeDtypeStruct((Npts, Cout), jnp.float32),
    )(x2, x3, x5.reshape(nb, 1, C5), *wflat)


def _knn_gather(x, nb, k=K):
    Npts, C = x.shape
    pp = Npts // nb
    xb = x.reshape(nb, pp, C)
    sq = jnp.sum(xb * xb, axis=-1)
    d = sq[:, :, None] + sq[:, None, :] - 2.0 * jnp.einsum('bic,bjc->bij', xb, xb)
    _, idx = jax.lax.top_k(-d, k)
    idx_t = jnp.transpose(idx, (2, 0, 1))
    bidx = jnp.arange(nb)[None, :, None]
    xj = xb[bidx, idx_t]
    return xj.reshape(k, Npts, C)


def _final_gn_layer(x, layer):
    W, b, g, be = layer
    x = jax.nn.relu(x @ W.T + b)
    Nr, C = x.shape
    mean = x.mean(axis=-1, keepdims=True)
    var = ((x - mean) ** 2).mean(axis=-1, keepdims=True)
    return (x - mean) * jax.lax.rsqrt(var + 1e-5) * g + be


def _forward(pos, params, nb):
    xj0 = _knn_gather(pos, nb)
    t1 = _edge_conv(pos, xj0, params['tconv1'], treefold=True)
    t3 = _mlp_pool(t1, params['tmlp1'], nb, treefold=True)
    t4 = _mlp_plain(t3, params['tmlp2'][:-1], treefold=True)
    t4 = _final_gn_layer(t4, params['tmlp2'][-1])
    pos2 = _transform(pos, t4, nb)
    x1 = _edge_conv(pos2, _knn_gather(pos2, nb), params['conv1'], treefold=True)
    x2 = _edge_conv(x1, _knn_gather(x1, nb), params['conv2'], treefold=True)
    x3 = _edge_conv(x2, _knn_gather(x2, nb), params['conv3'])
    x5 = _mlp_pool(x3, params['lin1'], nb)
    return _head(x2, x3, x5, params['head'],
                 params['lin2_W'], params['lin2_b'],
                 params['lin3_W'], params['lin3_b'], nb)


def kernel(pos, batch, params):
    del batch
    return _forward(pos, params, B)


# speedup vs baseline: 1.4950x; 1.4308x over previous
import functools
import jax
import jax.numpy as jnp
import numpy as np
from jax.experimental import pallas as pl
from jax.experimental.pallas import tpu as pltpu

B = 8
P = 2048
K = 30

_HI = jax.lax.Precision.HIGHEST


def _dot(a, b, precision=None):
    return jnp.dot(a, b, preferred_element_type=jnp.float32, precision=precision)


def _dot_hi(a, b):
    return jnp.dot(a, b, preferred_element_type=jnp.float32, precision=_HI)


def _gn_mats(C):
    groups = max(1, C // 16)
    gs = C // groups
    gid = np.arange(C) // gs
    M = np.zeros((C, groups), np.float32)
    M[np.arange(C), gid] = 1.0 / gs
    S = (M.T > 0).astype(np.float32)
    return jnp.asarray(M), jnp.asarray(S)


def _tower_params(layers):
    flat = []
    for (W, b, g, be) in layers:
        C = W.shape[0]
        M, S = _gn_mats(C)
        flat += [jnp.asarray(W.T), b[None, :], g[None, :], be[None, :], M, S]
    return flat


def _rotl(x, s):
    return jnp.concatenate([x[:, s:], x[:, :s]], axis=1)


def _rotr(x, s):
    return jnp.concatenate([x[:, -s:], x[:, :-s]], axis=1)


def _group_mean16(x, lane):
    s = x
    for sh in (8, 4, 2, 1):
        s = s + _rotl(s, sh)
    m = jnp.where(lane % 16 == 0, s * (1.0 / 16.0), 0.0)
    for sh in (1, 2, 4, 8):
        m = m + _rotr(m, sh)
    return m


def _apply_tower(x, wrefs, nl, treefold=False):
    for l in range(nl):
        Wt, b, g, be, M, S = (r[...] for r in wrefs[6 * l:6 * l + 6])
        x = _dot(x, Wt) + b
        x = jnp.maximum(x, 0.0)
        R, C = x.shape
        if treefold and C % 16 == 0:
            lane = jax.lax.broadcasted_iota(jnp.int32, (R, C), 1)
            mu = _group_mean16(x, lane)
            d = x - mu
            var = _group_mean16(d * d, lane)
            x = d * jax.lax.rsqrt(var + 1e-5) * g + be
        else:
            mu = _dot(_dot(x, M), S)
            d = x - mu
            var = _dot(_dot(d * d, M), S)
            x = d * jax.lax.rsqrt(var + 1e-5) * g + be
    return x


def _full_spec(w):
    return pl.BlockSpec(w.shape, lambda *i, nd=w.ndim: (0,) * nd)


def _edge_conv(x, xj_flat, layers, T=128, treefold=False):
    Npts, Cin = x.shape
    nl = len(layers)
    Cout = layers[-1][0].shape[0]
    wflat = _tower_params(layers)

    def body(*refs):
        xi_ref, xj_ref = refs[0], refs[1]
        wrefs = refs[2:2 + 6 * nl]
        o_ref = refs[-1]
        xi = xi_ref[...]
        xj = xj_ref[...].reshape(K * T, Cin)
        xi_rep = jnp.broadcast_to(xi[None], (K, T, Cin)).reshape(K * T, Cin)
        e = jnp.concatenate([xi_rep, xj - xi_rep], axis=-1)
        e = _apply_tower(e, wrefs, nl, treefold=treefold)
        acc = e[0:T]
        for k in range(1, K):
            acc = jnp.maximum(acc, e[k * T:(k + 1) * T])
        o_ref[...] = acc

    in_specs = [pl.BlockSpec((T, Cin), lambda i: (i, 0)),
                pl.BlockSpec((K, T, Cin), lambda i: (0, i, 0))]
    in_specs += [_full_spec(w) for w in wflat]
    return pl.pallas_call(
        body,
        grid=(Npts // T,),
        in_specs=in_specs,
        out_specs=pl.BlockSpec((T, Cout), lambda i: (i, 0)),
        out_shape=jax.ShapeDtypeStruct((Npts, Cout), jnp.float32),
        compiler_params=pltpu.CompilerParams(
            vmem_limit_bytes=100 * 1024 * 1024),
    )(x, xj_flat, *wflat)


def _mlp_pool(x, layers, nb, T=256, treefold=False):
    Npts, Cin = x.shape
    pp = Npts // nb
    nl = len(layers)
    Cout = layers[-1][0].shape[0]
    wflat = _tower_params(layers)

    def body(*refs):
        x_ref = refs[0]
        wrefs = refs[1:1 + 6 * nl]
        o_ref = refs[-1]
        y = _apply_tower(x_ref[...], wrefs, nl, treefold=treefold)
        m = jnp.max(y, axis=0, keepdims=True)[None]
        j = pl.program_id(1)

        @pl.when(j == 0)
        def _():
            o_ref[...] = m

        @pl.when(j != 0)
        def _():
            o_ref[...] = jnp.maximum(o_ref[...], m)

    in_specs = [pl.BlockSpec((T, Cin), lambda b, j: (b * (pp // T) + j, 0))]
    in_specs += [_full_spec(w) for w in wflat]
    out = pl.pallas_call(
        body,
        grid=(nb, pp // T),
        in_specs=in_specs,
        out_specs=pl.BlockSpec((1, 1, Cout), lambda b, j: (b, 0, 0)),
        out_shape=jax.ShapeDtypeStruct((nb, 1, Cout), jnp.float32),
        compiler_params=pltpu.CompilerParams(
            vmem_limit_bytes=100 * 1024 * 1024),
    )(x, *wflat)
    return out.reshape(nb, Cout)


def _mlp_plain(x, layers, treefold=False):
    R, Cin = x.shape
    nl = len(layers)
    Cout = layers[-1][0].shape[0]
    wflat = _tower_params(layers)

    def body(*refs):
        x_ref = refs[0]
        wrefs = refs[1:1 + 6 * nl]
        o_ref = refs[-1]
        o_ref[...] = _apply_tower(x_ref[...], wrefs, nl, treefold=treefold)

    in_specs = [_full_spec(x)] + [_full_spec(w) for w in wflat]
    return pl.pallas_call(
        body,
        grid=(1,),
        in_specs=in_specs,
        out_specs=pl.BlockSpec((R, Cout), lambda i: (0, 0)),
        out_shape=jax.ShapeDtypeStruct((R, Cout), jnp.float32),
    )(x, *wflat)


def _transform(pos, t4, nb):
    Npts = pos.shape[0]
    pp = Npts // nb
    pos3 = pos.reshape(nb, pp, 3)
    mats = t4.reshape(nb, 3, 3)

    def body(x_ref, m_ref, o_ref):
        o_ref[0] = _dot(x_ref[0], m_ref[0])

    out = pl.pallas_call(
        body,
        grid=(nb,),
        in_specs=[pl.BlockSpec((1, pp, 3), lambda b: (b, 0, 0)),
                  pl.BlockSpec((1, 3, 3), lambda b: (b, 0, 0))],
        out_specs=pl.BlockSpec((1, pp, 3), lambda b: (b, 0, 0)),
        out_shape=jax.ShapeDtypeStruct((nb, pp, 3), jnp.float32),
    )(pos3, mats)
    return out.reshape(Npts, 3)


def _head(x2, x3, x5, head_layers, lin2_W, lin2_b, lin3_W, lin3_b, nb, T=256):
    Npts, C2 = x2.shape
    pp = Npts // nb
    C3 = x3.shape[1]
    C5 = x5.shape[1]
    nl = len(head_layers)
    wflat = _tower_params(head_layers)
    wflat += [jnp.asarray(lin2_W.T), lin2_b[None, :],
              jnp.asarray(lin3_W.T), lin3_b[None, :]]
    Cout = lin3_W.shape[0]

    def body(*refs):
        x2_ref, x3_ref, x5_ref = refs[0], refs[1], refs[2]
        wrefs = refs[3:3 + 6 * nl]
        l2w, l2b, l3w, l3b = (r[...] for r in refs[3 + 6 * nl:3 + 6 * nl + 4])
        o_ref = refs[-1]
        x6 = jnp.broadcast_to(x5_ref[0], (T, C5))
        x = jnp.concatenate([x2_ref[...], x3_ref[...], x6], axis=-1)
        h = _apply_tower(x, wrefs, nl)
        h = _dot(h, l2w) + l2b
        h = _dot(h, l3w) + l3b
        m = jnp.max(h, axis=-1, keepdims=True)
        s = h - m
        lse = jnp.log(jnp.sum(jnp.exp(s), axis=-1, keepdims=True))
        o_ref[...] = s - lse

    row_spec = lambda C: pl.BlockSpec((T, C), lambda b, j: (b * (pp // T) + j, 0))
    in_specs = [row_spec(C2), row_spec(C3),
                pl.BlockSpec((1, 1, C5), lambda b, j: (b, 0, 0))]
    in_specs += [_full_spec(w) for w in wflat]
    return pl.pallas_call(
        body,
        grid=(nb, pp // T),
        in_specs=in_specs,
        out_specs=pl.BlockSpec((T, Cout), lambda b, j: (b * (pp // T) + j, 0)),
        out_shape=jax.ShapeDtypeStruct((Npts, Cout), jnp.float32),
    )(x2, x3, x5.reshape(nb, 1, C5), *wflat)


def _knn_gather(x, nb, k=K):
    Npts, C = x.shape
    pp = Npts // nb
    xb = x.reshape(nb, pp, C)
    sq = jnp.sum(xb * xb, axis=-1)
    d = sq[:, :, None] + sq[:, None, :] - 2.0 * jnp.einsum('bic,bjc->bij', xb, xb)
    _, idx = jax.lax.top_k(-d, k)
    idx_t = jnp.transpose(idx, (2, 0, 1))
    bidx = jnp.arange(nb)[None, :, None]
    xj = xb[bidx, idx_t]
    return xj.reshape(k, Npts, C)


def _final_gn_layer(x, layer):
    W, b, g, be = layer
    x = jax.nn.relu(x @ W.T + b)
    Nr, C = x.shape
    mean = x.mean(axis=-1, keepdims=True)
    var = ((x - mean) ** 2).mean(axis=-1, keepdims=True)
    return (x - mean) * jax.lax.rsqrt(var + 1e-5) * g + be


def _forward(pos, params, nb):
    xj0 = _knn_gather(pos, nb)
    t1 = _edge_conv(pos, xj0, params['tconv1'], treefold=True)
    t3 = _mlp_pool(t1, params['tmlp1'], nb, treefold=True)
    t4 = _mlp_plain(t3, params['tmlp2'][:-1], treefold=True)
    t4 = _final_gn_layer(t4, params['tmlp2'][-1])
    pos2 = _transform(pos, t4, nb)
    x1 = _edge_conv(pos2, _knn_gather(pos2, nb), params['conv1'], treefold=True)
    x2 = _edge_conv(x1, _knn_gather(x1, nb), params['conv2'], treefold=True)
    x3 = _edge_conv(x2, _knn_gather(x2, nb), params['conv3'])
    x5 = _mlp_pool(x3, params['lin1'], nb)
    return _head(x2, x3, x5, params['head'],
                 params['lin2_W'], params['lin2_b'],
                 params['lin3_W'], params['lin3_b'], nb)


def kernel(pos, batch, params):
    del batch
    return _forward(pos, params, B)


# speedup vs baseline: 1.5067x; 1.0078x over previous
import functools
import jax
import jax.numpy as jnp
import numpy as np
from jax.experimental import pallas as pl
from jax.experimental.pallas import tpu as pltpu

B = 8
P = 2048
K = 30

_HI = jax.lax.Precision.HIGHEST


def _dot(a, b, precision=None):
    return jnp.dot(a, b, preferred_element_type=jnp.float32, precision=precision)


def _dot_hi(a, b):
    return jnp.dot(a, b, preferred_element_type=jnp.float32, precision=_HI)


def _gn_mats(C):
    groups = max(1, C // 16)
    gs = C // groups
    gid = np.arange(C) // gs
    M = np.zeros((C, groups), np.float32)
    M[np.arange(C), gid] = 1.0 / gs
    S = (M.T > 0).astype(np.float32)
    return jnp.asarray(M), jnp.asarray(S)


def _tower_params(layers):
    flat = []
    for (W, b, g, be) in layers:
        C = W.shape[0]
        M, S = _gn_mats(C)
        flat += [jnp.asarray(W.T), b[None, :], g[None, :], be[None, :], M, S]
    return flat


def _rotl(x, s):
    return jnp.concatenate([x[:, s:], x[:, :s]], axis=1)


def _rotr(x, s):
    return jnp.concatenate([x[:, -s:], x[:, :-s]], axis=1)


def _group_mean16(x, lane):
    s = x
    for sh in (8, 4, 2, 1):
        s = s + _rotl(s, sh)
    m = jnp.where(lane % 16 == 0, s * (1.0 / 16.0), 0.0)
    for sh in (1, 2, 4, 8):
        m = m + _rotr(m, sh)
    return m


def _apply_tower(x, wrefs, nl, treefold=False):
    for l in range(nl):
        Wt, b, g, be, M, S = (r[...] for r in wrefs[6 * l:6 * l + 6])
        x = _dot(x, Wt) + b
        x = jnp.maximum(x, 0.0)
        R, C = x.shape
        if treefold and C % 16 == 0:
            lane = jax.lax.broadcasted_iota(jnp.int32, (R, C), 1)
            mu = _group_mean16(x, lane)
            d = x - mu
            var = _group_mean16(d * d, lane)
            x = d * jax.lax.rsqrt(var + 1e-5) * g + be
        else:
            mu = _dot(_dot(x, M), S)
            d = x - mu
            var = _dot(_dot(d * d, M), S)
            x = d * jax.lax.rsqrt(var + 1e-5) * g + be
    return x


def _full_spec(w):
    return pl.BlockSpec(w.shape, lambda *i, nd=w.ndim: (0,) * nd)


def _edge_conv(x, xj_flat, layers, T=256, treefold=False):
    Npts, Cin = x.shape
    nl = len(layers)
    Cout = layers[-1][0].shape[0]
    wflat = _tower_params(layers)

    def body(*refs):
        xi_ref, xj_ref = refs[0], refs[1]
        wrefs = refs[2:2 + 6 * nl]
        o_ref = refs[-1]
        xi = xi_ref[...]
        xj = xj_ref[...].reshape(K * T, Cin)
        xi_rep = jnp.broadcast_to(xi[None], (K, T, Cin)).reshape(K * T, Cin)
        e = jnp.concatenate([xi_rep, xj - xi_rep], axis=-1)
        e = _apply_tower(e, wrefs, nl, treefold=treefold)
        acc = e[0:T]
        for k in range(1, K):
            acc = jnp.maximum(acc, e[k * T:(k + 1) * T])
        o_ref[...] = acc

    in_specs = [pl.BlockSpec((T, Cin), lambda i: (i, 0)),
                pl.BlockSpec((K, T, Cin), lambda i: (0, i, 0))]
    in_specs += [_full_spec(w) for w in wflat]
    return pl.pallas_call(
        body,
        grid=(Npts // T,),
        in_specs=in_specs,
        out_specs=pl.BlockSpec((T, Cout), lambda i: (i, 0)),
        out_shape=jax.ShapeDtypeStruct((Npts, Cout), jnp.float32),
        compiler_params=pltpu.CompilerParams(
            vmem_limit_bytes=100 * 1024 * 1024),
    )(x, xj_flat, *wflat)


def _mlp_pool(x, layers, nb, T=512, treefold=False):
    Npts, Cin = x.shape
    pp = Npts // nb
    nl = len(layers)
    Cout = layers[-1][0].shape[0]
    wflat = _tower_params(layers)

    def body(*refs):
        x_ref = refs[0]
        wrefs = refs[1:1 + 6 * nl]
        o_ref = refs[-1]
        y = _apply_tower(x_ref[...], wrefs, nl, treefold=treefold)
        m = jnp.max(y, axis=0, keepdims=True)[None]
        j = pl.program_id(1)

        @pl.when(j == 0)
        def _():
            o_ref[...] = m

        @pl.when(j != 0)
        def _():
            o_ref[...] = jnp.maximum(o_ref[...], m)

    in_specs = [pl.BlockSpec((T, Cin), lambda b, j: (b * (pp // T) + j, 0))]
    in_specs += [_full_spec(w) for w in wflat]
    out = pl.pallas_call(
        body,
        grid=(nb, pp // T),
        in_specs=in_specs,
        out_specs=pl.BlockSpec((1, 1, Cout), lambda b, j: (b, 0, 0)),
        out_shape=jax.ShapeDtypeStruct((nb, 1, Cout), jnp.float32),
        compiler_params=pltpu.CompilerParams(
            vmem_limit_bytes=100 * 1024 * 1024),
    )(x, *wflat)
    return out.reshape(nb, Cout)


def _mlp_plain(x, layers, treefold=False):
    R, Cin = x.shape
    nl = len(layers)
    Cout = layers[-1][0].shape[0]
    wflat = _tower_params(layers)

    def body(*refs):
        x_ref = refs[0]
        wrefs = refs[1:1 + 6 * nl]
        o_ref = refs[-1]
        o_ref[...] = _apply_tower(x_ref[...], wrefs, nl, treefold=treefold)

    in_specs = [_full_spec(x)] + [_full_spec(w) for w in wflat]
    return pl.pallas_call(
        body,
        grid=(1,),
        in_specs=in_specs,
        out_specs=pl.BlockSpec((R, Cout), lambda i: (0, 0)),
        out_shape=jax.ShapeDtypeStruct((R, Cout), jnp.float32),
    )(x, *wflat)


def _transform(pos, t4, nb):
    Npts = pos.shape[0]
    pp = Npts // nb
    pos3 = pos.reshape(nb, pp, 3)
    mats = t4.reshape(nb, 3, 3)

    def body(x_ref, m_ref, o_ref):
        o_ref[0] = _dot(x_ref[0], m_ref[0])

    out = pl.pallas_call(
        body,
        grid=(nb,),
        in_specs=[pl.BlockSpec((1, pp, 3), lambda b: (b, 0, 0)),
                  pl.BlockSpec((1, 3, 3), lambda b: (b, 0, 0))],
        out_specs=pl.BlockSpec((1, pp, 3), lambda b: (b, 0, 0)),
        out_shape=jax.ShapeDtypeStruct((nb, pp, 3), jnp.float32),
    )(pos3, mats)
    return out.reshape(Npts, 3)


def _head(x2, x3, x5, head_layers, lin2_W, lin2_b, lin3_W, lin3_b, nb, T=512):
    Npts, C2 = x2.shape
    pp = Npts // nb
    C3 = x3.shape[1]
    C5 = x5.shape[1]
    nl = len(head_layers)
    wflat = _tower_params(head_layers)
    wflat += [jnp.asarray(lin2_W.T), lin2_b[None, :],
              jnp.asarray(lin3_W.T), lin3_b[None, :]]
    Cout = lin3_W.shape[0]

    def body(*refs):
        x2_ref, x3_ref, x5_ref = refs[0], refs[1], refs[2]
        wrefs = refs[3:3 + 6 * nl]
        l2w, l2b, l3w, l3b = (r[...] for r in refs[3 + 6 * nl:3 + 6 * nl + 4])
        o_ref = refs[-1]
        x6 = jnp.broadcast_to(x5_ref[0], (T, C5))
        x = jnp.concatenate([x2_ref[...], x3_ref[...], x6], axis=-1)
        h = _apply_tower(x, wrefs, nl)
        h = _dot(h, l2w) + l2b
        h = _dot(h, l3w) + l3b
        m = jnp.max(h, axis=-1, keepdims=True)
        s = h - m
        lse = jnp.log(jnp.sum(jnp.exp(s), axis=-1, keepdims=True))
        o_ref[...] = s - lse

    row_spec = lambda C: pl.BlockSpec((T, C), lambda b, j: (b * (pp // T) + j, 0))
    in_specs = [row_spec(C2), row_spec(C3),
                pl.BlockSpec((1, 1, C5), lambda b, j: (b, 0, 0))]
    in_specs += [_full_spec(w) for w in wflat]
    return pl.pallas_call(
        body,
        grid=(nb, pp // T),
        in_specs=in_specs,
        out_specs=pl.BlockSpec((T, Cout), lambda b, j: (b * (pp // T) + j, 0)),
        out_shape=jax.ShapeDtypeStruct((Npts, Cout), jnp.float32),
    )(x2, x3, x5.reshape(nb, 1, C5), *wflat)


def _knn_gather(x, nb, k=K):
    Npts, C = x.shape
    pp = Npts // nb
    xb = x.reshape(nb, pp, C)
    sq = jnp.sum(xb * xb, axis=-1)
    d = sq[:, :, None] + sq[:, None, :] - 2.0 * jnp.einsum('bic,bjc->bij', xb, xb)
    _, idx = jax.lax.top_k(-d, k)
    idx_t = jnp.transpose(idx, (2, 0, 1))
    bidx = jnp.arange(nb)[None, :, None]
    xj = xb[bidx, idx_t]
    return xj.reshape(k, Npts, C)


def _final_gn_layer(x, layer):
    W, b, g, be = layer
    x = jax.nn.relu(x @ W.T + b)
    Nr, C = x.shape
    mean = x.mean(axis=-1, keepdims=True)
    var = ((x - mean) ** 2).mean(axis=-1, keepdims=True)
    return (x - mean) * jax.lax.rsqrt(var + 1e-5) * g + be


def _forward(pos, params, nb):
    xj0 = _knn_gather(pos, nb)
    t1 = _edge_conv(pos, xj0, params['tconv1'], treefold=True)
    t3 = _mlp_pool(t1, params['tmlp1'], nb, treefold=True)
    t4 = _mlp_plain(t3, params['tmlp2'][:-1], treefold=True)
    t4 = _final_gn_layer(t4, params['tmlp2'][-1])
    pos2 = _transform(pos, t4, nb)
    x1 = _edge_conv(pos2, _knn_gather(pos2, nb), params['conv1'], treefold=True)
    x2 = _edge_conv(x1, _knn_gather(x1, nb), params['conv2'], treefold=True)
    x3 = _edge_conv(x2, _knn_gather(x2, nb), params['conv3'])
    x5 = _mlp_pool(x3, params['lin1'], nb)
    return _head(x2, x3, x5, params['head'],
                 params['lin2_W'], params['lin2_b'],
                 params['lin3_W'], params['lin3_b'], nb)


def kernel(pos, batch, params):
    del batch
    return _forward(pos, params, B)
